# Initial kernel scaffold; baseline (speedup 1.0000x reference)
import jax
import jax.numpy as jnp
from jax.experimental import pallas as pl


def kernel(inputs, labels, weights):
    raise NotImplementedError("write your pallas kernel here")


# speedup vs baseline: 6.3464x; 6.3464x over previous
import functools

import jax
import jax.numpy as jnp
from jax import lax
from jax.experimental import pallas as pl
from jax.experimental.pallas import tpu as pltpu
from jax.experimental.pallas import tpu_sc as plsc

_LANES = 16
_SUBCORES = 16
_BR = 1024


def _tc_losses_body(x_ref, lab_ref, w_ref, loss_ref):
    x = x_ref[...]
    lab = lab_ref[...]
    w = w_ref[...]
    m = jnp.max(x, axis=1, keepdims=True)
    lse = jnp.log(jnp.sum(jnp.exp(x - m), axis=1, keepdims=True)) + m
    onehot = lax.broadcasted_iota(jnp.int32, x.shape, 1) == lab
    picked = jnp.sum(jnp.where(onehot, x, 0.0), axis=1, keepdims=True)
    wl = jnp.sum(jnp.where(onehot, w, 0.0), axis=1, keepdims=True)
    loss_ref[...] = wl * (lse - picked)


def _sc_groupby_body(num_classes, chunk,
                     lab_hbm, loss_hbm, w_hbm, sums_hbm, outw_hbm,
                     lab_v, loss_v, hist_v, all_v, res_v, w_v, shared):
    c = num_classes
    wid = lax.axis_index("s")
    base = wid * chunk
    pltpu.sync_copy(lab_hbm.at[pl.ds(base, chunk)], lab_v)
    pltpu.sync_copy(loss_hbm.at[pl.ds(base, chunk)], loss_v)

    zeros = jnp.zeros((_LANES,), jnp.float32)
    for j in range(2 * c // _LANES):
        hist_v[pl.ds(j * _LANES, _LANES)] = zeros
    ones = jnp.ones((_LANES,), jnp.float32)

    def step(i, carry):
        labv = lab_v[pl.ds(i * _LANES, _LANES)]
        lossv = loss_v[pl.ds(i * _LANES, _LANES)]
        plsc.addupdate_scatter(hist_v, [labv], lossv)
        plsc.addupdate_scatter(hist_v, [labv + c], ones)
        return carry

    lax.fori_loop(0, chunk // _LANES, step, 0)

    pltpu.sync_copy(hist_v, shared.at[wid])
    plsc.subcore_barrier()

    @pl.when(wid == 0)
    def _():
        pltpu.sync_copy(w_hbm, w_v)
        pltpu.sync_copy(shared, all_v)
        for j in range(2 * c // _LANES):
            acc = jnp.zeros((_LANES,), jnp.float32)
            for k in range(_SUBCORES):
                acc = acc + all_v[k, pl.ds(j * _LANES, _LANES)]
            res_v[pl.ds(j * _LANES, _LANES)] = acc
        for j in range(c // _LANES):
            cnt = res_v[pl.ds(c + j * _LANES, _LANES)]
            wv = w_v[pl.ds(j * _LANES, _LANES)]
            res_v[pl.ds(c + j * _LANES, _LANES)] = cnt * wv
        pltpu.sync_copy(res_v.at[pl.ds(0, c)], sums_hbm)
        pltpu.sync_copy(res_v.at[pl.ds(c, c)], outw_hbm)


def kernel(inputs, labels, weights):
    n, c = inputs.shape
    grid = n // _BR

    losses = pl.pallas_call(
        _tc_losses_body,
        grid=(grid,),
        in_specs=[
            pl.BlockSpec((_BR, c), lambda i: (i, 0)),
            pl.BlockSpec((_BR, 1), lambda i: (i, 0)),
            pl.BlockSpec((1, c), lambda i: (0, 0)),
        ],
        out_specs=pl.BlockSpec((_BR, 1), lambda i: (i, 0)),
        out_shape=jax.ShapeDtypeStruct((n, 1), jnp.float32),
    )(inputs, labels.reshape(n, 1), weights.reshape(1, c))
    losses = losses.reshape(n)

    chunk = n // _SUBCORES
    mesh = plsc.VectorSubcoreMesh(
        core_axis_name="c", subcore_axis_name="s", num_cores=1)
    sc_call = functools.partial(
        pl.kernel,
        out_type=(jax.ShapeDtypeStruct((c,), jnp.float32),
                  jax.ShapeDtypeStruct((c,), jnp.float32)),
        mesh=mesh,
        scratch_types=[
            pltpu.VMEM((chunk,), jnp.int32),
            pltpu.VMEM((chunk,), jnp.float32),
            pltpu.VMEM((2 * c,), jnp.float32),
            pltpu.VMEM((_SUBCORES, 2 * c), jnp.float32),
            pltpu.VMEM((2 * c,), jnp.float32),
            pltpu.VMEM((c,), jnp.float32),
            pltpu.VMEM_SHARED((_SUBCORES, 2 * c), jnp.float32),
        ],
        compiler_params=pltpu.CompilerParams(needs_layout_passes=False),
    )(functools.partial(_sc_groupby_body, c, chunk))

    sum_by_class, out_weights = sc_call(labels, losses, weights)
    return (sum_by_class, out_weights)


# speedup vs baseline: 12.6238x; 1.9891x over previous
import functools

import jax
import jax.numpy as jnp
from jax import lax
from jax.experimental import pallas as pl
from jax.experimental.pallas import tpu as pltpu
from jax.experimental.pallas import tpu_sc as plsc

_LANES = 16
_SUBCORES = 16
_BR = 1024


def _tc_nll_body(x_ref, lab_ref, nll_ref):
    c = x_ref.shape[1]
    groups = x_ref.shape[0] // 128
    lab = lab_ref[...]
    for g in range(groups):
        xt = x_ref[pl.ds(g * 128, 128), :].T
        m = jnp.max(xt, axis=0, keepdims=True)
        s = jnp.sum(jnp.exp(xt - m), axis=0, keepdims=True)
        onehot = lax.broadcasted_iota(jnp.int32, (c, 128), 0) == lab[g:g + 1, :]
        picked = jnp.sum(jnp.where(onehot, xt, 0.0), axis=0, keepdims=True)
        nll_ref[pl.ds(g, 1), :] = jnp.log(s) + m - picked


def _sc_groupby_body(num_classes, chunk,
                     lab_hbm, loss_hbm, w_hbm, sums_hbm, outw_hbm,
                     lab_v, loss_v, hist_v, all_v, res_v, w_v, shared):
    c = num_classes
    wid = lax.axis_index("s")
    base = wid * chunk
    pltpu.sync_copy(lab_hbm.at[pl.ds(base, chunk)], lab_v)
    pltpu.sync_copy(loss_hbm.at[pl.ds(base, chunk)], loss_v)

    pltpu.sync_copy(w_hbm, w_v)

    zeros = jnp.zeros((_LANES,), jnp.float32)
    for j in range(2 * c // _LANES):
        hist_v[pl.ds(j * _LANES, _LANES)] = zeros
    ones = jnp.ones((_LANES,), jnp.float32)

    def step(i, carry):
        labv = lab_v[pl.ds(i * _LANES, _LANES)]
        nllv = loss_v[pl.ds(i * _LANES, _LANES)]
        wv = plsc.load_gather(w_v, [labv])
        plsc.addupdate_scatter(hist_v, [labv], wv * nllv)
        plsc.addupdate_scatter(hist_v, [labv + c], ones)
        return carry

    lax.fori_loop(0, chunk // _LANES, step, 0)

    pltpu.sync_copy(hist_v, shared.at[wid])
    plsc.subcore_barrier()

    @pl.when(wid == 0)
    def _():
        pltpu.sync_copy(shared, all_v)
        for j in range(2 * c // _LANES):
            acc = jnp.zeros((_LANES,), jnp.float32)
            for k in range(_SUBCORES):
                acc = acc + all_v[k, pl.ds(j * _LANES, _LANES)]
            res_v[pl.ds(j * _LANES, _LANES)] = acc
        for j in range(c // _LANES):
            cnt = res_v[pl.ds(c + j * _LANES, _LANES)]
            wv = w_v[pl.ds(j * _LANES, _LANES)]
            res_v[pl.ds(c + j * _LANES, _LANES)] = cnt * wv
        pltpu.sync_copy(res_v.at[pl.ds(0, c)], sums_hbm)
        pltpu.sync_copy(res_v.at[pl.ds(c, c)], outw_hbm)


def kernel(inputs, labels, weights):
    n, c = inputs.shape
    grid = n // _BR

    br_rows = _BR // 128
    nll = pl.pallas_call(
        _tc_nll_body,
        grid=(grid,),
        in_specs=[
            pl.BlockSpec((_BR, c), lambda i: (i, 0)),
            pl.BlockSpec((br_rows, 128), lambda i: (i, 0)),
        ],
        out_specs=pl.BlockSpec((br_rows, 128), lambda i: (i, 0)),
        out_shape=jax.ShapeDtypeStruct((n // 128, 128), jnp.float32),
    )(inputs, labels.reshape(n // 128, 128))
    losses = nll.reshape(n)

    chunk = n // _SUBCORES
    mesh = plsc.VectorSubcoreMesh(
        core_axis_name="c", subcore_axis_name="s", num_cores=1)
    sc_call = functools.partial(
        pl.kernel,
        out_type=(jax.ShapeDtypeStruct((c,), jnp.float32),
                  jax.ShapeDtypeStruct((c,), jnp.float32)),
        mesh=mesh,
        scratch_types=[
            pltpu.VMEM((chunk,), jnp.int32),
            pltpu.VMEM((chunk,), jnp.float32),
            pltpu.VMEM((2 * c,), jnp.float32),
            pltpu.VMEM((_SUBCORES, 2 * c), jnp.float32),
            pltpu.VMEM((2 * c,), jnp.float32),
            pltpu.VMEM((c,), jnp.float32),
            pltpu.VMEM_SHARED((_SUBCORES, 2 * c), jnp.float32),
        ],
        compiler_params=pltpu.CompilerParams(needs_layout_passes=False),
    )(functools.partial(_sc_groupby_body, c, chunk))

    sum_by_class, out_weights = sc_call(labels, losses, weights)
    return (sum_by_class, out_weights)


# speedup vs baseline: 22.0185x; 1.7442x over previous
import functools

import jax
import jax.numpy as jnp
from jax import lax
from jax.experimental import pallas as pl
from jax.experimental.pallas import tpu as pltpu
from jax.experimental.pallas import tpu_sc as plsc

_LANES = 16
_SUBCORES = 16
_BR = 4096


def _tc_nll_body(x_ref, lab_ref, nll_ref):
    c = x_ref.shape[1]
    groups = x_ref.shape[0] // 128
    lab = lab_ref[...]
    for g in range(groups):
        xt = x_ref[pl.ds(g * 128, 128), :].T
        m = jnp.max(xt, axis=0, keepdims=True)
        s = jnp.sum(jnp.exp(xt - m), axis=0, keepdims=True)
        onehot = lax.broadcasted_iota(jnp.int32, (c, 128), 0) == lab[g:g + 1, :]
        picked = jnp.sum(jnp.where(onehot, xt, 0.0), axis=0, keepdims=True)
        nll_ref[pl.ds(g, 1), :] = jnp.log(s) + m - picked


def _sc_groupby_body(num_classes, chunk,
                     lab_hbm, loss_hbm, w_hbm, sums_hbm, outw_hbm,
                     lab_v, loss_v, hist_v, all_v, res_v, w_v, shared):
    c = num_classes
    wid = lax.axis_index("s")
    base = wid * chunk
    pltpu.sync_copy(lab_hbm.at[pl.ds(base, chunk)], lab_v)
    pltpu.sync_copy(loss_hbm.at[pl.ds(base, chunk)], loss_v)

    pltpu.sync_copy(w_hbm, w_v)

    zeros = jnp.zeros((_LANES,), jnp.float32)
    for j in range(2 * c // _LANES):
        hist_v[pl.ds(j * _LANES, _LANES)] = zeros
    ones = jnp.ones((_LANES,), jnp.float32)

    def step(i, carry):
        labv = lab_v[pl.ds(i * _LANES, _LANES)]
        nllv = loss_v[pl.ds(i * _LANES, _LANES)]
        wv = plsc.load_gather(w_v, [labv])
        plsc.addupdate_scatter(hist_v, [labv], wv * nllv)
        plsc.addupdate_scatter(hist_v, [labv + c], ones)
        return carry

    lax.fori_loop(0, chunk // _LANES, step, 0)

    pltpu.sync_copy(hist_v, shared.at[wid])
    plsc.subcore_barrier()

    @pl.when(wid == 0)
    def _():
        pltpu.sync_copy(shared, all_v)
        for j in range(2 * c // _LANES):
            acc = jnp.zeros((_LANES,), jnp.float32)
            for k in range(_SUBCORES):
                acc = acc + all_v[k, pl.ds(j * _LANES, _LANES)]
            res_v[pl.ds(j * _LANES, _LANES)] = acc
        for j in range(c // _LANES):
            cnt = res_v[pl.ds(c + j * _LANES, _LANES)]
            wv = w_v[pl.ds(j * _LANES, _LANES)]
            res_v[pl.ds(c + j * _LANES, _LANES)] = cnt * wv
        pltpu.sync_copy(res_v.at[pl.ds(0, c)], sums_hbm)
        pltpu.sync_copy(res_v.at[pl.ds(c, c)], outw_hbm)


def kernel(inputs, labels, weights):
    n, c = inputs.shape
    grid = n // _BR

    br_rows = _BR // 128
    nll = pl.pallas_call(
        _tc_nll_body,
        grid=(grid,),
        in_specs=[
            pl.BlockSpec((_BR, c), lambda i: (i, 0)),
            pl.BlockSpec((br_rows, 128), lambda i: (i, 0)),
        ],
        out_specs=pl.BlockSpec((br_rows, 128), lambda i: (i, 0)),
        out_shape=jax.ShapeDtypeStruct((n // 128, 128), jnp.float32),
    )(inputs, labels.reshape(n // 128, 128))
    losses = nll.reshape(n)

    chunk = n // _SUBCORES
    mesh = plsc.VectorSubcoreMesh(
        core_axis_name="c", subcore_axis_name="s", num_cores=1)
    sc_call = functools.partial(
        pl.kernel,
        out_type=(jax.ShapeDtypeStruct((c,), jnp.float32),
                  jax.ShapeDtypeStruct((c,), jnp.float32)),
        mesh=mesh,
        scratch_types=[
            pltpu.VMEM((chunk,), jnp.int32),
            pltpu.VMEM((chunk,), jnp.float32),
            pltpu.VMEM((2 * c,), jnp.float32),
            pltpu.VMEM((_SUBCORES, 2 * c), jnp.float32),
            pltpu.VMEM((2 * c,), jnp.float32),
            pltpu.VMEM((c,), jnp.float32),
            pltpu.VMEM_SHARED((_SUBCORES, 2 * c), jnp.float32),
        ],
        compiler_params=pltpu.CompilerParams(needs_layout_passes=False),
    )(functools.partial(_sc_groupby_body, c, chunk))

    sum_by_class, out_weights = sc_call(labels, losses, weights)
    return (sum_by_class, out_weights)


# speedup vs baseline: 25.2169x; 1.1453x over previous
import functools

import jax
import jax.numpy as jnp
from jax import lax
from jax.experimental import pallas as pl
from jax.experimental.pallas import tpu as pltpu
from jax.experimental.pallas import tpu_sc as plsc

_LANES = 16
_SUBCORES = 16
_BR = 8192


def _tc_nll_body(x_ref, lab_ref, nll_ref):
    c = x_ref.shape[1]
    groups = x_ref.shape[0] // 128
    lab = lab_ref[...]
    for g in range(groups):
        xt = x_ref[pl.ds(g * 128, 128), :].T
        m = jnp.max(xt, axis=0, keepdims=True)
        s = jnp.sum(jnp.exp(xt - m), axis=0, keepdims=True)
        onehot = lax.broadcasted_iota(jnp.int32, (c, 128), 0) == lab[g:g + 1, :]
        picked = jnp.sum(jnp.where(onehot, xt, 0.0), axis=0, keepdims=True)
        nll_ref[pl.ds(g, 1), :] = jnp.log(s) + m - picked


def _sc_groupby_body(num_classes, chunk,
                     lab_hbm, loss_hbm, w_hbm, sums_hbm, outw_hbm,
                     lab_v, loss_v, hist_v, all_v, res_v, w_v, shared):
    c = num_classes
    wid = lax.axis_index("s")
    base = wid * chunk
    pltpu.sync_copy(lab_hbm.at[pl.ds(base, chunk)], lab_v)
    pltpu.sync_copy(loss_hbm.at[pl.ds(base, chunk)], loss_v)

    pltpu.sync_copy(w_hbm, w_v)

    zeros = jnp.zeros((_LANES,), jnp.float32)
    for j in range(2 * c // _LANES):
        hist_v[pl.ds(j * _LANES, _LANES)] = zeros
    ones = jnp.ones((_LANES,), jnp.float32)

    def step(i, carry):
        labv = lab_v[pl.ds(i * _LANES, _LANES)]
        nllv = loss_v[pl.ds(i * _LANES, _LANES)]
        wv = plsc.load_gather(w_v, [labv])
        plsc.addupdate_scatter(hist_v, [labv], wv * nllv)
        plsc.addupdate_scatter(hist_v, [labv + c], ones)
        return carry

    lax.fori_loop(0, chunk // _LANES, step, 0)

    pltpu.sync_copy(hist_v, shared.at[wid])
    plsc.subcore_barrier()

    @pl.when(wid == 0)
    def _():
        pltpu.sync_copy(shared, all_v)
        for j in range(2 * c // _LANES):
            acc = jnp.zeros((_LANES,), jnp.float32)
            for k in range(_SUBCORES):
                acc = acc + all_v[k, pl.ds(j * _LANES, _LANES)]
            res_v[pl.ds(j * _LANES, _LANES)] = acc
        for j in range(c // _LANES):
            cnt = res_v[pl.ds(c + j * _LANES, _LANES)]
            wv = w_v[pl.ds(j * _LANES, _LANES)]
            res_v[pl.ds(c + j * _LANES, _LANES)] = cnt * wv
        pltpu.sync_copy(res_v.at[pl.ds(0, c)], sums_hbm)
        pltpu.sync_copy(res_v.at[pl.ds(c, c)], outw_hbm)


def kernel(inputs, labels, weights):
    n, c = inputs.shape
    grid = n // _BR

    br_rows = _BR // 128
    nll = pl.pallas_call(
        _tc_nll_body,
        grid=(grid,),
        in_specs=[
            pl.BlockSpec((_BR, c), lambda i: (i, 0)),
            pl.BlockSpec((br_rows, 128), lambda i: (i, 0)),
        ],
        out_specs=pl.BlockSpec((br_rows, 128), lambda i: (i, 0)),
        out_shape=jax.ShapeDtypeStruct((n // 128, 128), jnp.float32),
    )(inputs, labels.reshape(n // 128, 128))
    losses = nll.reshape(n)

    chunk = n // _SUBCORES
    mesh = plsc.VectorSubcoreMesh(
        core_axis_name="c", subcore_axis_name="s", num_cores=1)
    sc_call = functools.partial(
        pl.kernel,
        out_type=(jax.ShapeDtypeStruct((c,), jnp.float32),
                  jax.ShapeDtypeStruct((c,), jnp.float32)),
        mesh=mesh,
        scratch_types=[
            pltpu.VMEM((chunk,), jnp.int32),
            pltpu.VMEM((chunk,), jnp.float32),
            pltpu.VMEM((2 * c,), jnp.float32),
            pltpu.VMEM((_SUBCORES, 2 * c), jnp.float32),
            pltpu.VMEM((2 * c,), jnp.float32),
            pltpu.VMEM((c,), jnp.float32),
            pltpu.VMEM_SHARED((_SUBCORES, 2 * c), jnp.float32),
        ],
        compiler_params=pltpu.CompilerParams(needs_layout_passes=False),
    )(functools.partial(_sc_groupby_body, c, chunk))

    sum_by_class, out_weights = sc_call(labels, losses, weights)
    return (sum_by_class, out_weights)


# speedup vs baseline: 26.6588x; 1.0572x over previous
import functools

import jax
import jax.numpy as jnp
from jax import lax
from jax.experimental import pallas as pl
from jax.experimental.pallas import tpu as pltpu
from jax.experimental.pallas import tpu_sc as plsc

_LANES = 16
_SUBCORES = 16
_BR = 16384


def _tc_nll_body(x_ref, lab_ref, nll_ref):
    c = x_ref.shape[1]
    groups = x_ref.shape[0] // 128
    lab = lab_ref[...]
    for g in range(groups):
        xt = x_ref[pl.ds(g * 128, 128), :].T
        m = jnp.max(xt, axis=0, keepdims=True)
        s = jnp.sum(jnp.exp(xt - m), axis=0, keepdims=True)
        onehot = lax.broadcasted_iota(jnp.int32, (c, 128), 0) == lab[g:g + 1, :]
        picked = jnp.sum(jnp.where(onehot, xt, 0.0), axis=0, keepdims=True)
        nll_ref[pl.ds(g, 1), :] = jnp.log(s) + m - picked


def _sc_groupby_body(num_classes, chunk,
                     lab_hbm, loss_hbm, w_hbm, sums_hbm, outw_hbm,
                     lab_v, loss_v, hist_v, all_v, res_v, w_v, shared):
    c = num_classes
    wid = lax.axis_index("s")
    base = wid * chunk
    pltpu.sync_copy(lab_hbm.at[pl.ds(base, chunk)], lab_v)
    pltpu.sync_copy(loss_hbm.at[pl.ds(base, chunk)], loss_v)

    pltpu.sync_copy(w_hbm, w_v)

    zeros = jnp.zeros((_LANES,), jnp.float32)
    for j in range(2 * c // _LANES):
        hist_v[pl.ds(j * _LANES, _LANES)] = zeros
    ones = jnp.ones((_LANES,), jnp.float32)

    def step(i, carry):
        labv = lab_v[pl.ds(i * _LANES, _LANES)]
        nllv = loss_v[pl.ds(i * _LANES, _LANES)]
        wv = plsc.load_gather(w_v, [labv])
        plsc.addupdate_scatter(hist_v, [labv], wv * nllv)
        plsc.addupdate_scatter(hist_v, [labv + c], ones)
        return carry

    lax.fori_loop(0, chunk // _LANES, step, 0)

    pltpu.sync_copy(hist_v, shared.at[wid])
    plsc.subcore_barrier()

    @pl.when(wid == 0)
    def _():
        pltpu.sync_copy(shared, all_v)
        for j in range(2 * c // _LANES):
            acc = jnp.zeros((_LANES,), jnp.float32)
            for k in range(_SUBCORES):
                acc = acc + all_v[k, pl.ds(j * _LANES, _LANES)]
            res_v[pl.ds(j * _LANES, _LANES)] = acc
        for j in range(c // _LANES):
            cnt = res_v[pl.ds(c + j * _LANES, _LANES)]
            wv = w_v[pl.ds(j * _LANES, _LANES)]
            res_v[pl.ds(c + j * _LANES, _LANES)] = cnt * wv
        pltpu.sync_copy(res_v.at[pl.ds(0, c)], sums_hbm)
        pltpu.sync_copy(res_v.at[pl.ds(c, c)], outw_hbm)


def kernel(inputs, labels, weights):
    n, c = inputs.shape
    grid = n // _BR

    br_rows = _BR // 128
    nll = pl.pallas_call(
        _tc_nll_body,
        grid=(grid,),
        in_specs=[
            pl.BlockSpec((_BR, c), lambda i: (i, 0)),
            pl.BlockSpec((br_rows, 128), lambda i: (i, 0)),
        ],
        out_specs=pl.BlockSpec((br_rows, 128), lambda i: (i, 0)),
        out_shape=jax.ShapeDtypeStruct((n // 128, 128), jnp.float32),
    )(inputs, labels.reshape(n // 128, 128))
    losses = nll.reshape(n)

    chunk = n // _SUBCORES
    mesh = plsc.VectorSubcoreMesh(
        core_axis_name="c", subcore_axis_name="s", num_cores=1)
    sc_call = functools.partial(
        pl.kernel,
        out_type=(jax.ShapeDtypeStruct((c,), jnp.float32),
                  jax.ShapeDtypeStruct((c,), jnp.float32)),
        mesh=mesh,
        scratch_types=[
            pltpu.VMEM((chunk,), jnp.int32),
            pltpu.VMEM((chunk,), jnp.float32),
            pltpu.VMEM((2 * c,), jnp.float32),
            pltpu.VMEM((_SUBCORES, 2 * c), jnp.float32),
            pltpu.VMEM((2 * c,), jnp.float32),
            pltpu.VMEM((c,), jnp.float32),
            pltpu.VMEM_SHARED((_SUBCORES, 2 * c), jnp.float32),
        ],
        compiler_params=pltpu.CompilerParams(needs_layout_passes=False),
    )(functools.partial(_sc_groupby_body, c, chunk))

    sum_by_class, out_weights = sc_call(labels, losses, weights)
    return (sum_by_class, out_weights)


# speedup vs baseline: 26.6990x; 1.0015x over previous
import functools

import jax
import jax.numpy as jnp
from jax import lax
from jax.experimental import pallas as pl
from jax.experimental.pallas import tpu as pltpu
from jax.experimental.pallas import tpu_sc as plsc

_LANES = 16
_SUBCORES = 16
_BR = 32768


def _tc_nll_body(x_ref, lab_ref, nll_ref):
    c = x_ref.shape[1]
    groups = x_ref.shape[0] // 128
    lab = lab_ref[...]
    for g in range(groups):
        xt = x_ref[pl.ds(g * 128, 128), :].T
        m = jnp.max(xt, axis=0, keepdims=True)
        s = jnp.sum(jnp.exp(xt - m), axis=0, keepdims=True)
        onehot = lax.broadcasted_iota(jnp.int32, (c, 128), 0) == lab[g:g + 1, :]
        picked = jnp.sum(jnp.where(onehot, xt, 0.0), axis=0, keepdims=True)
        nll_ref[pl.ds(g, 1), :] = jnp.log(s) + m - picked


def _sc_groupby_body(num_classes, chunk,
                     lab_hbm, loss_hbm, w_hbm, sums_hbm, outw_hbm,
                     lab_v, loss_v, hist_v, all_v, res_v, w_v, shared):
    c = num_classes
    wid = lax.axis_index("s")
    base = wid * chunk
    pltpu.sync_copy(lab_hbm.at[pl.ds(base, chunk)], lab_v)
    pltpu.sync_copy(loss_hbm.at[pl.ds(base, chunk)], loss_v)

    pltpu.sync_copy(w_hbm, w_v)

    zeros = jnp.zeros((_LANES,), jnp.float32)
    for j in range(2 * c // _LANES):
        hist_v[pl.ds(j * _LANES, _LANES)] = zeros
    ones = jnp.ones((_LANES,), jnp.float32)

    def step(i, carry):
        labv = lab_v[pl.ds(i * _LANES, _LANES)]
        nllv = loss_v[pl.ds(i * _LANES, _LANES)]
        wv = plsc.load_gather(w_v, [labv])
        plsc.addupdate_scatter(hist_v, [labv], wv * nllv)
        plsc.addupdate_scatter(hist_v, [labv + c], ones)
        return carry

    lax.fori_loop(0, chunk // _LANES, step, 0)

    pltpu.sync_copy(hist_v, shared.at[wid])
    plsc.subcore_barrier()

    @pl.when(wid == 0)
    def _():
        pltpu.sync_copy(shared, all_v)
        for j in range(2 * c // _LANES):
            acc = jnp.zeros((_LANES,), jnp.float32)
            for k in range(_SUBCORES):
                acc = acc + all_v[k, pl.ds(j * _LANES, _LANES)]
            res_v[pl.ds(j * _LANES, _LANES)] = acc
        for j in range(c // _LANES):
            cnt = res_v[pl.ds(c + j * _LANES, _LANES)]
            wv = w_v[pl.ds(j * _LANES, _LANES)]
            res_v[pl.ds(c + j * _LANES, _LANES)] = cnt * wv
        pltpu.sync_copy(res_v.at[pl.ds(0, c)], sums_hbm)
        pltpu.sync_copy(res_v.at[pl.ds(c, c)], outw_hbm)


def kernel(inputs, labels, weights):
    n, c = inputs.shape
    grid = n // _BR

    br_rows = _BR // 128
    nll = pl.pallas_call(
        _tc_nll_body,
        grid=(grid,),
        in_specs=[
            pl.BlockSpec((_BR, c), lambda i: (i, 0)),
            pl.BlockSpec((br_rows, 128), lambda i: (i, 0)),
        ],
        out_specs=pl.BlockSpec((br_rows, 128), lambda i: (i, 0)),
        out_shape=jax.ShapeDtypeStruct((n // 128, 128), jnp.float32),
    )(inputs, labels.reshape(n // 128, 128))
    losses = nll.reshape(n)

    chunk = n // _SUBCORES
    mesh = plsc.VectorSubcoreMesh(
        core_axis_name="c", subcore_axis_name="s", num_cores=1)
    sc_call = functools.partial(
        pl.kernel,
        out_type=(jax.ShapeDtypeStruct((c,), jnp.float32),
                  jax.ShapeDtypeStruct((c,), jnp.float32)),
        mesh=mesh,
        scratch_types=[
            pltpu.VMEM((chunk,), jnp.int32),
            pltpu.VMEM((chunk,), jnp.float32),
            pltpu.VMEM((2 * c,), jnp.float32),
            pltpu.VMEM((_SUBCORES, 2 * c), jnp.float32),
            pltpu.VMEM((2 * c,), jnp.float32),
            pltpu.VMEM((c,), jnp.float32),
            pltpu.VMEM_SHARED((_SUBCORES, 2 * c), jnp.float32),
        ],
        compiler_params=pltpu.CompilerParams(needs_layout_passes=False),
    )(functools.partial(_sc_groupby_body, c, chunk))

    sum_by_class, out_weights = sc_call(labels, losses, weights)
    return (sum_by_class, out_weights)


# speedup vs baseline: 26.7289x; 1.0011x over previous
import functools

import jax
import jax.numpy as jnp
from jax import lax
from jax.experimental import pallas as pl
from jax.experimental.pallas import tpu as pltpu
from jax.experimental.pallas import tpu_sc as plsc

_LANES = 16
_SUBCORES = 16
_BR = 32768


def _tc_nll_body(x_ref, lab_ref, nll_ref):
    c = x_ref.shape[1]
    groups = x_ref.shape[0] // 128
    lab = lab_ref[...]
    for g in range(groups):
        xt = x_ref[pl.ds(g * 128, 128), :].T
        m = jnp.max(xt, axis=0, keepdims=True)
        s = jnp.sum(jnp.exp(xt - m), axis=0, keepdims=True)
        onehot = lax.broadcasted_iota(jnp.int32, (c, 128), 0) == lab[g:g + 1, :]
        picked = jnp.sum(jnp.where(onehot, xt, 0.0), axis=0, keepdims=True)
        nll_ref[pl.ds(g, 1), :] = jnp.log(s) + m - picked


def _sc_groupby_body(num_classes, rows,
                     lab_hbm, loss_hbm, w_hbm, sums_hbm, outw_hbm,
                     lab_v, loss_v, hist_v, all_v, res_v, w_v, shared):
    c = num_classes
    wid = lax.axis_index("s")
    base = wid * rows
    pltpu.sync_copy(lab_hbm.at[pl.ds(base, rows), :], lab_v)
    pltpu.sync_copy(loss_hbm.at[pl.ds(base, rows), :], loss_v)

    pltpu.sync_copy(w_hbm, w_v)

    zeros = jnp.zeros((_LANES,), jnp.float32)
    for j in range(2 * c // _LANES):
        hist_v[pl.ds(j * _LANES, _LANES)] = zeros
    ones = jnp.ones((_LANES,), jnp.float32)

    def step(r, carry):
        for j in range(128 // _LANES):
            labv = lab_v[r, pl.ds(j * _LANES, _LANES)]
            nllv = loss_v[r, pl.ds(j * _LANES, _LANES)]
            wv = plsc.load_gather(w_v, [labv])
            plsc.addupdate_scatter(hist_v, [labv], wv * nllv)
            plsc.addupdate_scatter(hist_v, [labv + c], ones)
        return carry

    lax.fori_loop(0, rows, step, 0)

    pltpu.sync_copy(hist_v, shared.at[wid])
    plsc.subcore_barrier()

    @pl.when(wid == 0)
    def _():
        pltpu.sync_copy(shared, all_v)
        for j in range(2 * c // _LANES):
            acc = jnp.zeros((_LANES,), jnp.float32)
            for k in range(_SUBCORES):
                acc = acc + all_v[k, pl.ds(j * _LANES, _LANES)]
            res_v[pl.ds(j * _LANES, _LANES)] = acc
        for j in range(c // _LANES):
            cnt = res_v[pl.ds(c + j * _LANES, _LANES)]
            wv = w_v[pl.ds(j * _LANES, _LANES)]
            res_v[pl.ds(c + j * _LANES, _LANES)] = cnt * wv
        pltpu.sync_copy(res_v.at[pl.ds(0, c)], sums_hbm)
        pltpu.sync_copy(res_v.at[pl.ds(c, c)], outw_hbm)


def kernel(inputs, labels, weights):
    n, c = inputs.shape
    grid = n // _BR
    lab2d = labels.reshape(n // 128, 128)

    br_rows = _BR // 128
    nll = pl.pallas_call(
        _tc_nll_body,
        grid=(grid,),
        in_specs=[
            pl.BlockSpec((_BR, c), lambda i: (i, 0)),
            pl.BlockSpec((br_rows, 128), lambda i: (i, 0)),
        ],
        out_specs=pl.BlockSpec((br_rows, 128), lambda i: (i, 0)),
        out_shape=jax.ShapeDtypeStruct((n // 128, 128), jnp.float32),
    )(inputs, lab2d)

    rows = (n // 128) // _SUBCORES
    mesh = plsc.VectorSubcoreMesh(
        core_axis_name="c", subcore_axis_name="s", num_cores=1)
    sc_call = functools.partial(
        pl.kernel,
        out_type=(jax.ShapeDtypeStruct((c,), jnp.float32),
                  jax.ShapeDtypeStruct((c,), jnp.float32)),
        mesh=mesh,
        scratch_types=[
            pltpu.VMEM((rows, 128), jnp.int32),
            pltpu.VMEM((rows, 128), jnp.float32),
            pltpu.VMEM((2 * c,), jnp.float32),
            pltpu.VMEM((_SUBCORES, 2 * c), jnp.float32),
            pltpu.VMEM((2 * c,), jnp.float32),
            pltpu.VMEM((c,), jnp.float32),
            pltpu.VMEM_SHARED((_SUBCORES, 2 * c), jnp.float32),
        ],
        compiler_params=pltpu.CompilerParams(needs_layout_passes=False),
    )(functools.partial(_sc_groupby_body, c, rows))

    sum_by_class, out_weights = sc_call(lab2d, nll, weights)
    return (sum_by_class, out_weights)


# speedup vs baseline: 27.3877x; 1.0246x over previous
import functools

import jax
import jax.numpy as jnp
from jax import lax
from jax.experimental import pallas as pl
from jax.experimental.pallas import tpu as pltpu
from jax.experimental.pallas import tpu_sc as plsc

_LANES = 16
_SUBCORES = 16
_BR = 32768


def _tc_nll_body(x_ref, lab_ref, nll_ref):
    c = x_ref.shape[1]
    groups = x_ref.shape[0] // 128
    lab = lab_ref[...]
    ones_row = jnp.ones((1, c), jnp.float32)
    for g in range(groups):
        xt = x_ref[pl.ds(g * 128, 128), :].T
        m = jnp.max(xt, axis=0, keepdims=True)
        e = jnp.exp(xt - m)
        s = jnp.dot(ones_row, e, preferred_element_type=jnp.float32)
        onehot = lax.broadcasted_iota(jnp.int32, (c, 128), 0) == lab[g:g + 1, :]
        masked = jnp.where(onehot, xt, 0.0)
        picked = jnp.dot(ones_row, masked, preferred_element_type=jnp.float32)
        nll_ref[pl.ds(g, 1), :] = jnp.log(s) + m - picked


def _sc_groupby_body(num_classes, rows,
                     lab_hbm, loss_hbm, w_hbm, sums_hbm, outw_hbm,
                     lab_v, loss_v, hist_v, all_v, res_v, w_v, shared):
    c = num_classes
    wid = lax.axis_index("s")
    base = wid * rows
    pltpu.sync_copy(lab_hbm.at[pl.ds(base, rows), :], lab_v)
    pltpu.sync_copy(loss_hbm.at[pl.ds(base, rows), :], loss_v)

    pltpu.sync_copy(w_hbm, w_v)

    zeros = jnp.zeros((_LANES,), jnp.float32)
    for j in range(2 * c // _LANES):
        hist_v[pl.ds(j * _LANES, _LANES)] = zeros
    ones = jnp.ones((_LANES,), jnp.float32)

    def step(r, carry):
        for j in range(128 // _LANES):
            labv = lab_v[r, pl.ds(j * _LANES, _LANES)]
            nllv = loss_v[r, pl.ds(j * _LANES, _LANES)]
            wv = plsc.load_gather(w_v, [labv])
            plsc.addupdate_scatter(hist_v, [labv], wv * nllv)
            plsc.addupdate_scatter(hist_v, [labv + c], ones)
        return carry

    lax.fori_loop(0, rows, step, 0)

    pltpu.sync_copy(hist_v, shared.at[wid])
    plsc.subcore_barrier()

    @pl.when(wid == 0)
    def _():
        pltpu.sync_copy(shared, all_v)
        for j in range(2 * c // _LANES):
            acc = jnp.zeros((_LANES,), jnp.float32)
            for k in range(_SUBCORES):
                acc = acc + all_v[k, pl.ds(j * _LANES, _LANES)]
            res_v[pl.ds(j * _LANES, _LANES)] = acc
        for j in range(c // _LANES):
            cnt = res_v[pl.ds(c + j * _LANES, _LANES)]
            wv = w_v[pl.ds(j * _LANES, _LANES)]
            res_v[pl.ds(c + j * _LANES, _LANES)] = cnt * wv
        pltpu.sync_copy(res_v.at[pl.ds(0, c)], sums_hbm)
        pltpu.sync_copy(res_v.at[pl.ds(c, c)], outw_hbm)


def kernel(inputs, labels, weights):
    n, c = inputs.shape
    grid = n // _BR
    lab2d = labels.reshape(n // 128, 128)

    br_rows = _BR // 128
    nll = pl.pallas_call(
        _tc_nll_body,
        grid=(grid,),
        in_specs=[
            pl.BlockSpec((_BR, c), lambda i: (i, 0)),
            pl.BlockSpec((br_rows, 128), lambda i: (i, 0)),
        ],
        out_specs=pl.BlockSpec((br_rows, 128), lambda i: (i, 0)),
        out_shape=jax.ShapeDtypeStruct((n // 128, 128), jnp.float32),
    )(inputs, lab2d)

    rows = (n // 128) // _SUBCORES
    mesh = plsc.VectorSubcoreMesh(
        core_axis_name="c", subcore_axis_name="s", num_cores=1)
    sc_call = functools.partial(
        pl.kernel,
        out_type=(jax.ShapeDtypeStruct((c,), jnp.float32),
                  jax.ShapeDtypeStruct((c,), jnp.float32)),
        mesh=mesh,
        scratch_types=[
            pltpu.VMEM((rows, 128), jnp.int32),
            pltpu.VMEM((rows, 128), jnp.float32),
            pltpu.VMEM((2 * c,), jnp.float32),
            pltpu.VMEM((_SUBCORES, 2 * c), jnp.float32),
            pltpu.VMEM((2 * c,), jnp.float32),
            pltpu.VMEM((c,), jnp.float32),
            pltpu.VMEM_SHARED((_SUBCORES, 2 * c), jnp.float32),
        ],
        compiler_params=pltpu.CompilerParams(needs_layout_passes=False),
    )(functools.partial(_sc_groupby_body, c, rows))

    sum_by_class, out_weights = sc_call(lab2d, nll, weights)
    return (sum_by_class, out_weights)
